# Initial kernel scaffold; baseline (speedup 1.0000x reference)
import jax
import jax.numpy as jnp
from jax.experimental import pallas as pl


def kernel(input_ids, wte):
    raise NotImplementedError("write your pallas kernel here")


# speedup vs baseline: 1.6219x; 1.6219x over previous
import functools

import jax
import jax.numpy as jnp
from jax import lax
from jax.experimental import pallas as pl
from jax.experimental.pallas import tpu as pltpu
from jax.experimental.pallas import tpu_sc as plsc

N_EMBD = 1024
ROWS = 4 * 8192
NW = 32
ROWS_PER_W = ROWS // NW
CHUNK = 64
NCHUNK = ROWS_PER_W // CHUNK

_mesh = plsc.VectorSubcoreMesh(core_axis_name="c", subcore_axis_name="s")


@functools.partial(
    pl.kernel,
    out_type=jax.ShapeDtypeStruct((ROWS, N_EMBD), jnp.float32),
    mesh=_mesh,
    scratch_types=[
        pltpu.VMEM((ROWS_PER_W,), jnp.int32),
        pltpu.VMEM((CHUNK, N_EMBD), jnp.float32),
        pltpu.SemaphoreType.DMA,
    ],
)
def _embed_sc(ids_hbm, table_hbm, out_hbm, idx_v, rows_v, sem):
    wid = lax.axis_index("s") * 2 + lax.axis_index("c")
    base = wid * ROWS_PER_W
    pltpu.sync_copy(ids_hbm.at[pl.ds(base, ROWS_PER_W)], idx_v)

    @pl.loop(0, NCHUNK)
    def _chunk(c):
        off = c * CHUNK
        pltpu.async_copy(
            table_hbm.at[idx_v.at[pl.ds(off, CHUNK)]], rows_v, sem
        ).wait()
        pltpu.sync_copy(rows_v, out_hbm.at[pl.ds(base + off, CHUNK)])


def kernel(input_ids, wte):
    ids = input_ids.reshape(-1).astype(jnp.int32)
    flat = _embed_sc(ids, wte)
    return flat.reshape(input_ids.shape[0], input_ids.shape[1], N_EMBD)


# speedup vs baseline: 1.7271x; 1.0648x over previous
import functools

import jax
import jax.numpy as jnp
from jax import lax
from jax.experimental import pallas as pl
from jax.experimental.pallas import tpu as pltpu
from jax.experimental.pallas import tpu_sc as plsc

N_EMBD = 1024
ROWS = 4 * 8192
NW = 32
ROWS_PER_W = ROWS // NW
CHUNK = 32
NCHUNK = ROWS_PER_W // CHUNK

_mesh = plsc.VectorSubcoreMesh(core_axis_name="c", subcore_axis_name="s")


@functools.partial(
    pl.kernel,
    out_type=jax.ShapeDtypeStruct((ROWS, N_EMBD), jnp.float32),
    mesh=_mesh,
    scratch_types=[
        pltpu.VMEM((ROWS_PER_W,), jnp.int32),
        pltpu.VMEM((CHUNK, N_EMBD), jnp.float32),
        pltpu.VMEM((CHUNK, N_EMBD), jnp.float32),
        pltpu.SemaphoreType.DMA,
        pltpu.SemaphoreType.DMA,
        pltpu.SemaphoreType.DMA,
        pltpu.SemaphoreType.DMA,
    ],
)
def _embed_sc(ids_hbm, table_hbm, out_hbm, idx_v, buf0, buf1,
              gsem0, gsem1, wsem0, wsem1):
    wid = lax.axis_index("s") * 2 + lax.axis_index("c")
    base = wid * ROWS_PER_W
    pltpu.sync_copy(ids_hbm.at[pl.ds(base, ROWS_PER_W)], idx_v)

    bufs = (buf0, buf1)
    gsems = (gsem0, gsem1)
    wsems = (wsem0, wsem1)

    gd = [None] * NCHUNK
    wd = [None] * NCHUNK
    for i in range(NCHUNK):
        b = i % 2
        if i >= 2:
            wd[i - 2].wait()
        gd[i] = pltpu.async_copy(
            table_hbm.at[idx_v.at[pl.ds(i * CHUNK, CHUNK)]], bufs[b], gsems[b]
        )
        if i >= 1:
            gd[i - 1].wait()
            wd[i - 1] = pltpu.async_copy(
                bufs[1 - b], out_hbm.at[pl.ds(base + (i - 1) * CHUNK, CHUNK)],
                wsems[1 - b],
            )
    last = NCHUNK - 1
    gd[last].wait()
    wd[last] = pltpu.async_copy(
        bufs[last % 2], out_hbm.at[pl.ds(base + last * CHUNK, CHUNK)],
        wsems[last % 2],
    )
    wd[last - 1].wait()
    wd[last].wait()


def kernel(input_ids, wte):
    ids = input_ids.reshape(-1).astype(jnp.int32)
    flat = _embed_sc(ids, wte)
    return flat.reshape(input_ids.shape[0], input_ids.shape[1], N_EMBD)


# speedup vs baseline: 1.7291x; 1.0012x over previous
import functools

import jax
import jax.numpy as jnp
from jax import lax
from jax.experimental import pallas as pl
from jax.experimental.pallas import tpu as pltpu
from jax.experimental.pallas import tpu_sc as plsc

N_EMBD = 1024
ROWS = 4 * 8192
NW = 32
ROWS_PER_W = ROWS // NW
CHUNK = 16
NCHUNK = ROWS_PER_W // CHUNK
NB = 7
G = 3

_mesh = plsc.VectorSubcoreMesh(core_axis_name="c", subcore_axis_name="s")


@functools.partial(
    pl.kernel,
    out_type=jax.ShapeDtypeStruct((ROWS, N_EMBD), jnp.float32),
    mesh=_mesh,
    scratch_types=[
        pltpu.VMEM((ROWS_PER_W,), jnp.int32),
        pltpu.VMEM((NB, CHUNK, N_EMBD), jnp.float32),
        pltpu.SemaphoreType.DMA,
        pltpu.SemaphoreType.DMA,
        pltpu.SemaphoreType.DMA,
        pltpu.SemaphoreType.DMA,
        pltpu.SemaphoreType.DMA,
        pltpu.SemaphoreType.DMA,
        pltpu.SemaphoreType.DMA,
        pltpu.SemaphoreType.DMA,
        pltpu.SemaphoreType.DMA,
        pltpu.SemaphoreType.DMA,
        pltpu.SemaphoreType.DMA,
        pltpu.SemaphoreType.DMA,
        pltpu.SemaphoreType.DMA,
        pltpu.SemaphoreType.DMA,
    ],
)
def _embed_sc(ids_hbm, table_hbm, out_hbm, idx_v, bufs, *sems):
    gsems = sems[:NB]
    wsems = sems[NB:]
    wid = lax.axis_index("s") * 2 + lax.axis_index("c")
    base = wid * ROWS_PER_W
    pltpu.sync_copy(ids_hbm.at[pl.ds(base, ROWS_PER_W)], idx_v)

    gd = [None] * NCHUNK
    wd = [None] * NCHUNK
    for i in range(NCHUNK):
        b = i % NB
        if i >= NB:
            wd[i - NB].wait()
        gd[i] = pltpu.async_copy(
            table_hbm.at[idx_v.at[pl.ds(i * CHUNK, CHUNK)]], bufs.at[b],
            gsems[b],
        )
        if i >= G:
            j = i - G
            gd[j].wait()
            wd[j] = pltpu.async_copy(
                bufs.at[j % NB], out_hbm.at[pl.ds(base + j * CHUNK, CHUNK)],
                wsems[j % NB],
            )
    for j in range(NCHUNK - G, NCHUNK):
        gd[j].wait()
        wd[j] = pltpu.async_copy(
            bufs.at[j % NB], out_hbm.at[pl.ds(base + j * CHUNK, CHUNK)],
            wsems[j % NB],
        )
    for j in range(NCHUNK - NB, NCHUNK):
        wd[j].wait()


def kernel(input_ids, wte):
    ids = input_ids.reshape(-1).astype(jnp.int32)
    flat = _embed_sc(ids, wte)
    return flat.reshape(input_ids.shape[0], input_ids.shape[1], N_EMBD)
